# baseline (device time: 12954 ns/iter reference)
import jax
import jax.numpy as jnp
from jax import lax
from jax.experimental import pallas as pl
from jax.experimental.pallas import tpu as pltpu

N_DEV = 4
B, Sq, Skv, Hq, Dh = 2, 128, 512, 4, 64
D_MODEL = 512
S_PER = Skv // N_DEV
WINDOW = 128
SCALE = 0.125
HD = Hq * Dh
PW = HD + 2 * Hq


def kernel(x, Wq, K_ext, V_ext, Wo):
    x2 = x.reshape(B * Sq, D_MODEL)
    k2 = K_ext.reshape(B, S_PER, HD)
    v2 = V_ext.reshape(B, S_PER, HD)

    def body(x_ref, wq_ref, k_ref, v_ref, wo_ref, out_ref,
             recv0, recv1, send_sems, recv_sems):
        my = lax.axis_index("i")

        barrier_sem = pltpu.get_barrier_semaphore()
        for k in range(1, N_DEV):
            pl.semaphore_signal(
                barrier_sem, inc=1,
                device_id=((my + k) % N_DEV,),
                device_id_type=pl.DeviceIdType.MESH,
            )

        def compute_partial(dst, masked):
            if masked:
                qi = lax.broadcasted_iota(jnp.int32, (Sq, S_PER), 0)
                kj = lax.broadcasted_iota(jnp.int32, (Sq, S_PER), 1)
                mask = kj <= qi
            q_all = jnp.dot(x_ref[...], wq_ref[...],
                            preferred_element_type=jnp.float32)
            for b in range(B):
                q_b = q_all[b * Sq:(b + 1) * Sq]
                k_b = k_ref[b]
                v_b = v_ref[b]
                us, ms, ls = [], [], []
                for h in range(Hq):
                    q_bh = q_b[:, h * Dh:(h + 1) * Dh]
                    k_bh = k_b[:, h * Dh:(h + 1) * Dh]
                    v_bh = v_b[:, h * Dh:(h + 1) * Dh]
                    s = lax.dot_general(
                        q_bh, k_bh, (((1,), (1,)), ((), ())),
                        preferred_element_type=jnp.float32,
                    ) * SCALE
                    if masked:
                        s = jnp.where(mask, s, -1e9)
                    m = jnp.max(s, axis=1, keepdims=True)
                    e = jnp.exp(s - m)
                    l = jnp.sum(e, axis=1, keepdims=True)
                    u = jnp.dot(e, v_bh,
                                preferred_element_type=jnp.float32)
                    us.append(u)
                    ms.append(m)
                    ls.append(l)
                dst[b, :, 0:HD] = jnp.concatenate(us, axis=1).astype(jnp.bfloat16)
                dst[b, :, HD:PW] = jnp.concatenate(ms + ls, axis=1).astype(jnp.bfloat16)

        @pl.when(my == 0)
        def _():
            compute_partial(recv0, masked=False)

        @pl.when(my == 1)
        def _():
            compute_partial(recv1, masked=True)

        pl.semaphore_wait(barrier_sem, N_DEV - 1)

        @pl.when(my == 0)
        def _():
            for i, tgt in enumerate([2, 1, 3]):
                pltpu.make_async_remote_copy(
                    src_ref=recv0, dst_ref=recv0,
                    send_sem=send_sems.at[i], recv_sem=recv_sems.at[0],
                    device_id=(tgt,), device_id_type=pl.DeviceIdType.MESH,
                ).start()

        @pl.when(my == 1)
        def _():
            for i, tgt in enumerate([3, 2, 0]):
                pltpu.make_async_remote_copy(
                    src_ref=recv1, dst_ref=recv1,
                    send_sem=send_sems.at[i], recv_sem=recv_sems.at[1],
                    device_id=(tgt,), device_id_type=pl.DeviceIdType.MESH,
                ).start()

        @pl.when(my != 0)
        def _():
            pltpu.make_async_remote_copy(
                src_ref=recv0, dst_ref=recv0,
                send_sem=send_sems.at[0], recv_sem=recv_sems.at[0],
                device_id=(0,), device_id_type=pl.DeviceIdType.MESH,
            ).wait_recv()

        @pl.when(my != 1)
        def _():
            pltpu.make_async_remote_copy(
                src_ref=recv1, dst_ref=recv1,
                send_sem=send_sems.at[0], recv_sem=recv_sems.at[1],
                device_id=(1,), device_id_type=pl.DeviceIdType.MESH,
            ).wait_recv()

        ctx_rows = []
        for b in range(B):
            ctxs = []
            for h in range(Hq):
                u0 = recv0[b, :, h * Dh:(h + 1) * Dh].astype(jnp.float32)
                u1 = recv1[b, :, h * Dh:(h + 1) * Dh].astype(jnp.float32)
                m0 = recv0[b, :, HD + h:HD + h + 1].astype(jnp.float32)
                m1 = recv1[b, :, HD + h:HD + h + 1].astype(jnp.float32)
                l0 = recv0[b, :, HD + Hq + h:HD + Hq + h + 1].astype(jnp.float32)
                l1 = recv1[b, :, HD + Hq + h:HD + Hq + h + 1].astype(jnp.float32)
                m = jnp.maximum(m0, m1)
                a0 = jnp.exp(m0 - m)
                a1 = jnp.exp(m1 - m)
                den = a0 * l0 + a1 * l1
                ctxs.append((a0 * u0 + a1 * u1) / den)
            ctx_rows.append(jnp.concatenate(ctxs, axis=1))
        ctx_all = jnp.concatenate(ctx_rows, axis=0)
        out_ref[...] = jnp.dot(ctx_all, wo_ref[...],
                               preferred_element_type=jnp.float32)

        @pl.when(my < 2)
        def _():
            for i in range(3):
                pltpu.make_async_remote_copy(
                    src_ref=recv0, dst_ref=recv0,
                    send_sem=send_sems.at[i], recv_sem=recv_sems.at[0],
                    device_id=(0,), device_id_type=pl.DeviceIdType.MESH,
                ).wait_send()

    out = pl.pallas_call(
        body,
        out_shape=jax.ShapeDtypeStruct((B * Sq, D_MODEL), jnp.float32),
        in_specs=[pl.BlockSpec(memory_space=pltpu.VMEM)] * 5,
        out_specs=pl.BlockSpec(memory_space=pltpu.VMEM),
        scratch_shapes=[
            pltpu.VMEM((B, Sq, PW), jnp.bfloat16),
            pltpu.VMEM((B, Sq, PW), jnp.bfloat16),
            pltpu.SemaphoreType.DMA((3,)),
            pltpu.SemaphoreType.DMA((2,)),
        ],
        compiler_params=pltpu.CompilerParams(collective_id=0),
    )(x2, Wq, k2, v2, Wo)
    return out.reshape(B, Sq, D_MODEL)


# device time: 11863 ns/iter; 1.0920x vs baseline; 1.0920x over previous
import jax
import jax.numpy as jnp
from jax import lax
from jax.experimental import pallas as pl
from jax.experimental.pallas import tpu as pltpu

N_DEV = 4
B, Sq, Skv, Hq, Dh = 2, 128, 512, 4, 64
D_MODEL = 512
S_PER = Skv // N_DEV
WINDOW = 128
SCALE = 0.125
HD = Hq * Dh
PW = HD + 2 * Hq


def kernel(x, Wq, K_ext, V_ext, Wo):
    def body(x_ref, wq_ref, k_ref, v_ref, wo_ref, out_ref,
             recv0, recv1, send_sems, recv_sems):
        my = lax.axis_index("i")

        barrier_sem = pltpu.get_barrier_semaphore()
        for k in range(1, N_DEV):
            pl.semaphore_signal(
                barrier_sem, inc=1,
                device_id=((my + k) % N_DEV,),
                device_id_type=pl.DeviceIdType.MESH,
            )

        def compute_partial_b(dst, b, masked):
            q_b = jnp.dot(x_ref[b], wq_ref[...],
                          preferred_element_type=jnp.float32)
            us, ms, ls = [], [], []
            for h in range(Hq):
                q_bh = q_b[:, h * Dh:(h + 1) * Dh]
                k_bh = k_ref[b, :, h, :]
                v_bh = v_ref[b, :, h, :]
                s = lax.dot_general(
                    q_bh, k_bh, (((1,), (1,)), ((), ())),
                    preferred_element_type=jnp.float32,
                ) * SCALE
                if masked:
                    qi = lax.broadcasted_iota(jnp.int32, (Sq, S_PER), 0)
                    kj = lax.broadcasted_iota(jnp.int32, (Sq, S_PER), 1)
                    s = jnp.where(kj <= qi, s, -1e9)
                m = jnp.max(s, axis=1, keepdims=True)
                e = jnp.exp(s - m)
                l = jnp.sum(e, axis=1, keepdims=True)
                u = jnp.dot(e, v_bh, preferred_element_type=jnp.float32)
                us.append(u)
                ms.append(m)
                ls.append(l)
            dst[b, :, 0:HD] = jnp.concatenate(us, axis=1).astype(jnp.bfloat16)
            dst[b, :, HD:PW] = jnp.concatenate(ms + ls, axis=1).astype(jnp.bfloat16)

        def send_chunk(src, b, targets, recv_sem_idx):
            for i, tgt in enumerate(targets):
                pltpu.make_async_remote_copy(
                    src_ref=src.at[b], dst_ref=src.at[b],
                    send_sem=send_sems.at[b * 3 + i],
                    recv_sem=recv_sems.at[recv_sem_idx * 2 + b],
                    device_id=(tgt,), device_id_type=pl.DeviceIdType.MESH,
                ).start()

        @pl.when(my == 0)
        def _():
            compute_partial_b(recv0, 0, masked=False)

        @pl.when(my == 1)
        def _():
            compute_partial_b(recv1, 0, masked=True)

        pl.semaphore_wait(barrier_sem, N_DEV - 1)

        @pl.when(my == 0)
        def _():
            send_chunk(recv0, 0, [2, 1, 3], 0)
            compute_partial_b(recv0, 1, masked=False)
            send_chunk(recv0, 1, [2, 1, 3], 0)

        @pl.when(my == 1)
        def _():
            send_chunk(recv1, 0, [3, 2, 0], 1)
            compute_partial_b(recv1, 1, masked=True)
            send_chunk(recv1, 1, [3, 2, 0], 1)

        for b in range(B):
            @pl.when(my != 0)
            def _():
                pltpu.make_async_remote_copy(
                    src_ref=recv0.at[b], dst_ref=recv0.at[b],
                    send_sem=send_sems.at[0], recv_sem=recv_sems.at[0 * 2 + b],
                    device_id=(0,), device_id_type=pl.DeviceIdType.MESH,
                ).wait_recv()

            @pl.when(my != 1)
            def _():
                pltpu.make_async_remote_copy(
                    src_ref=recv1.at[b], dst_ref=recv1.at[b],
                    send_sem=send_sems.at[0], recv_sem=recv_sems.at[1 * 2 + b],
                    device_id=(1,), device_id_type=pl.DeviceIdType.MESH,
                ).wait_recv()

            ctxs = []
            for h in range(Hq):
                u0 = recv0[b, :, h * Dh:(h + 1) * Dh].astype(jnp.float32)
                u1 = recv1[b, :, h * Dh:(h + 1) * Dh].astype(jnp.float32)
                m0 = recv0[b, :, HD + h:HD + h + 1].astype(jnp.float32)
                m1 = recv1[b, :, HD + h:HD + h + 1].astype(jnp.float32)
                l0 = recv0[b, :, HD + Hq + h:HD + Hq + h + 1].astype(jnp.float32)
                l1 = recv1[b, :, HD + Hq + h:HD + Hq + h + 1].astype(jnp.float32)
                m = jnp.maximum(m0, m1)
                a0 = jnp.exp(m0 - m)
                a1 = jnp.exp(m1 - m)
                den = a0 * l0 + a1 * l1
                ctxs.append((a0 * u0 + a1 * u1) / den)
            ctx_b = jnp.concatenate(ctxs, axis=1)
            out_ref[b] = jnp.dot(ctx_b, wo_ref[...],
                                 preferred_element_type=jnp.float32)

        @pl.when(my < 2)
        def _():
            for i in range(2 * 3):
                pltpu.make_async_remote_copy(
                    src_ref=recv0.at[0], dst_ref=recv0.at[0],
                    send_sem=send_sems.at[i], recv_sem=recv_sems.at[0],
                    device_id=(0,), device_id_type=pl.DeviceIdType.MESH,
                ).wait_send()

    return pl.pallas_call(
        body,
        out_shape=jax.ShapeDtypeStruct((B, Sq, D_MODEL), jnp.float32),
        in_specs=[pl.BlockSpec(memory_space=pltpu.VMEM)] * 5,
        out_specs=pl.BlockSpec(memory_space=pltpu.VMEM),
        scratch_shapes=[
            pltpu.VMEM((B, Sq, PW), jnp.bfloat16),
            pltpu.VMEM((B, Sq, PW), jnp.bfloat16),
            pltpu.SemaphoreType.DMA((2 * 3,)),
            pltpu.SemaphoreType.DMA((2 * 2,)),
        ],
        compiler_params=pltpu.CompilerParams(collective_id=0),
    )(x, Wq, K_ext, V_ext, Wo)
